# Initial kernel scaffold; baseline (speedup 1.0000x reference)
import jax
import jax.numpy as jnp
from jax.experimental import pallas as pl


def kernel(x, adj, W, a_src, a_dst):
    raise NotImplementedError("write your pallas kernel here")


# speedup vs baseline: 2.7638x; 2.7638x over previous
import jax
import jax.numpy as jnp
from jax.experimental import pallas as pl
from jax.experimental.pallas import tpu as pltpu

_B, _N, _IN, _HID, _NH = 4, 1024, 256, 256, 8
_DH = _HID // _NH


def _gat_kernel(x_ref, adj_ref, w_ref, asrc_ref, adst_ref, out_ref):
    xb = x_ref[0]
    h = jnp.dot(xb, w_ref[...], preferred_element_type=jnp.float32)
    e_src = jnp.dot(h, asrc_ref[...], preferred_element_type=jnp.float32)
    e_dst = jnp.dot(h, adst_ref[...], preferred_element_type=jnp.float32)
    e_dst_t = e_dst.T
    mask = adj_ref[0] > 0
    neg = jnp.float32(-9e15)
    outs = []
    for i in range(_NH):
        v = e_src[:, i:i + 1] + e_dst_t[i:i + 1, :]
        v = jnp.where(v > 0, v, 0.2 * v)
        v = jnp.where(mask, v, neg)
        m = jnp.max(v, axis=1, keepdims=True)
        p = jnp.exp(v - m)
        s = jnp.sum(p, axis=1, keepdims=True)
        alpha = p / s
        outs.append(jnp.dot(alpha, h[:, i * _DH:(i + 1) * _DH],
                            preferred_element_type=jnp.float32))
    hcat = jnp.concatenate(outs, axis=1)
    out_ref[0] = jnp.maximum(hcat, 0.0) + xb


def _build_call(interpret=False):
    grid = (_B,)
    return pl.pallas_call(
        _gat_kernel,
        grid=grid,
        in_specs=[
            pl.BlockSpec((1, _N, _IN), lambda b: (b, 0, 0)),
            pl.BlockSpec((1, _N, _N), lambda b: (b, 0, 0)),
            pl.BlockSpec((_IN, _HID), lambda b: (0, 0)),
            pl.BlockSpec((_HID, _NH), lambda b: (0, 0)),
            pl.BlockSpec((_HID, _NH), lambda b: (0, 0)),
        ],
        out_specs=pl.BlockSpec((1, _N, _HID), lambda b: (b, 0, 0)),
        out_shape=jax.ShapeDtypeStruct((_B, _N, _HID), jnp.float32),
        compiler_params=pltpu.CompilerParams(
            dimension_semantics=("arbitrary",),
        ),
        interpret=interpret,
    )


def kernel(x, adj, W, a_src, a_dst):
    Wfull = jnp.transpose(W, (1, 0, 2)).reshape(_IN, _HID)
    eye = jnp.eye(_NH, dtype=jnp.float32)
    Asrc = (a_src[:, :, None] * eye[:, None, :]).reshape(_HID, _NH)
    Adst = (a_dst[:, :, None] * eye[:, None, :]).reshape(_HID, _NH)
    return _build_call()(x, adj, Wfull, Asrc, Adst)


# speedup vs baseline: 3.0268x; 1.0952x over previous
import jax
import jax.numpy as jnp
from jax.experimental import pallas as pl
from jax.experimental.pallas import tpu as pltpu

_B, _N, _IN, _HID, _NH = 4, 1024, 256, 256, 8
_DH = _HID // _NH


def _gat_kernel(x_ref, adj_ref, w_ref, asrc_ref, adst_ref, out_ref):
    xb = x_ref[0]
    h = jnp.dot(xb, w_ref[...], preferred_element_type=jnp.float32)
    e_src = jnp.dot(h, asrc_ref[...], preferred_element_type=jnp.float32)
    e_dst = jnp.dot(h, adst_ref[...], preferred_element_type=jnp.float32)
    e_dst_t = e_dst.T
    mask = adj_ref[0] > 0
    neg = jnp.float32(-9e15)
    hb = h.astype(jnp.bfloat16)
    outs = []
    for i in range(_NH):
        v = e_src[:, i:i + 1] + e_dst_t[i:i + 1, :]
        v = jnp.maximum(v, 0.2 * v)
        v = jnp.where(mask, v, neg)
        m = jnp.max(v, axis=1, keepdims=True)
        p = jnp.exp(v - m)
        s = jnp.sum(p, axis=1, keepdims=True)
        o = jnp.dot(p.astype(jnp.bfloat16), hb[:, i * _DH:(i + 1) * _DH],
                    preferred_element_type=jnp.float32)
        outs.append(o / s)
    hcat = jnp.concatenate(outs, axis=1)
    out_ref[0] = jnp.maximum(hcat, 0.0) + xb


def _build_call(interpret=False):
    grid = (_B,)
    return pl.pallas_call(
        _gat_kernel,
        grid=grid,
        in_specs=[
            pl.BlockSpec((1, _N, _IN), lambda b: (b, 0, 0)),
            pl.BlockSpec((1, _N, _N), lambda b: (b, 0, 0)),
            pl.BlockSpec((_IN, _HID), lambda b: (0, 0)),
            pl.BlockSpec((_HID, _NH), lambda b: (0, 0)),
            pl.BlockSpec((_HID, _NH), lambda b: (0, 0)),
        ],
        out_specs=pl.BlockSpec((1, _N, _HID), lambda b: (b, 0, 0)),
        out_shape=jax.ShapeDtypeStruct((_B, _N, _HID), jnp.float32),
        compiler_params=pltpu.CompilerParams(
            dimension_semantics=("arbitrary",),
        ),
        interpret=interpret,
    )


def kernel(x, adj, W, a_src, a_dst):
    Wfull = jnp.transpose(W, (1, 0, 2)).reshape(_IN, _HID)
    eye = jnp.eye(_NH, dtype=jnp.float32)
    Asrc = (a_src[:, :, None] * eye[:, None, :]).reshape(_HID, _NH)
    Adst = (a_dst[:, :, None] * eye[:, None, :]).reshape(_HID, _NH)
    return _build_call()(x, adj, Wfull, Asrc, Adst)
